# Initial kernel scaffold; baseline (speedup 1.0000x reference)
import jax
import jax.numpy as jnp
from jax.experimental import pallas as pl


def kernel(x, adj, lt):
    raise NotImplementedError("write your pallas kernel here")


# speedup vs baseline: 1.3998x; 1.3998x over previous
import jax
import jax.numpy as jnp
from jax.experimental import pallas as pl

_TILE = 8000


def _concat_kernel(lt_ref, x_ref, o_ref):
    d = lt_ref.shape[1]
    o_ref[:, :d] = lt_ref[...]
    o_ref[:, d:] = x_ref[...]


def kernel(x, adj, lt):
    del adj
    n, d_lt = lt.shape
    d_x = x.shape[1]
    return pl.pallas_call(
        _concat_kernel,
        grid=(n // _TILE,),
        in_specs=[
            pl.BlockSpec((_TILE, d_lt), lambda i: (i, 0)),
            pl.BlockSpec((_TILE, d_x), lambda i: (i, 0)),
        ],
        out_specs=pl.BlockSpec((_TILE, d_lt + d_x), lambda i: (i, 0)),
        out_shape=jax.ShapeDtypeStruct((n, d_lt + d_x), lt.dtype),
    )(lt, x)


# speedup vs baseline: 1.4005x; 1.0005x over previous
import jax
import jax.numpy as jnp
from jax.experimental import pallas as pl

_TILE = 20000


def _concat_kernel(lt_ref, x_ref, o_ref):
    d = lt_ref.shape[1]
    o_ref[:, :d] = lt_ref[...]
    o_ref[:, d:] = x_ref[...]


def kernel(x, adj, lt):
    del adj
    n, d_lt = lt.shape
    d_x = x.shape[1]
    return pl.pallas_call(
        _concat_kernel,
        grid=(n // _TILE,),
        in_specs=[
            pl.BlockSpec((_TILE, d_lt), lambda i: (i, 0)),
            pl.BlockSpec((_TILE, d_x), lambda i: (i, 0)),
        ],
        out_specs=pl.BlockSpec((_TILE, d_lt + d_x), lambda i: (i, 0)),
        out_shape=jax.ShapeDtypeStruct((n, d_lt + d_x), lt.dtype),
    )(lt, x)
